# baseline (speedup 1.0000x reference)
import jax
import jax.numpy as jnp
from jax.experimental import pallas as pl
from jax.experimental.pallas import tpu as pltpu

_CHUNK = 512
_HALF = 256
_SL = 256
_NORM_PERSISTENT = 1024.0


def _qkproj_kernel(q_ref, k_ref, m_acc, out_ref, n_acc):
    i = pl.program_id(0)

    @pl.when(i == 0)
    def _init():
        n_acc[0, 0] = _NORM_PERSISTENT

    dim = q_ref.shape[1]
    n_sl = dim // _SL
    cdims = (((1,), (1,)), ((), ()))
    rdims = (((1,), (0,)), ((), ()))

    q = q_ref[...]
    k = k_ref[...]
    q0, q1 = q[:_HALF], q[_HALF:]
    k0, k1 = k[:_HALF], k[_HALF:]

    out = jax.lax.dot_general(q, m_acc[...], rdims,
                              preferred_element_type=jnp.float32)

    s00 = jax.lax.dot_general(q0, k0, cdims,
                              preferred_element_type=jnp.float32)
    s10 = jax.lax.dot_general(q1, k0, cdims,
                              preferred_element_type=jnp.float32)
    s11 = jax.lax.dot_general(q1, k1, cdims,
                              preferred_element_type=jnp.float32)
    row = jax.lax.broadcasted_iota(jnp.int32, (_HALF, _HALF), 0)
    col = jax.lax.broadcasted_iota(jnp.int32, (_HALF, _HALF), 1)
    causal = (col <= row)
    s00 = jnp.where(causal, s00, 0.0)
    s11 = jnp.where(causal, s11, 0.0)

    ss = jnp.sum(k * k, axis=1, keepdims=True)
    causal_f = causal.astype(jnp.float32)
    ss0, ss1 = ss[:_HALF], ss[_HALF:]
    csum0 = jnp.dot(causal_f, ss0, preferred_element_type=jnp.float32)
    csum1 = (jnp.sum(ss0) + jnp.dot(causal_f, ss1,
                                    preferred_element_type=jnp.float32))
    base = n_acc[0, 0]
    norms0 = jnp.maximum(base + csum0, 1e-8)
    norms1 = jnp.maximum(base + csum1, 1e-8)
    n_acc[0, 0] = base + jnp.sum(ss)

    d0 = jax.lax.dot_general(s00, k0, rdims,
                             preferred_element_type=jnp.float32)
    d1 = (jax.lax.dot_general(s10, k0, rdims,
                              preferred_element_type=jnp.float32)
          + jax.lax.dot_general(s11, k1, rdims,
                                preferred_element_type=jnp.float32))
    out_ref[:_HALF, :] = (out[:_HALF] + d0) * (1.0 / norms0)
    out_ref[_HALF:, :] = (out[_HALF:] + d1) * (1.0 / norms1)

    ks_parts = [k[:, a * _SL:(a + 1) * _SL] for a in range(n_sl)]
    for a in range(n_sl):
        sla = pl.ds(a * _SL, _SL)
        for b in range(a, n_sl):
            slb = pl.ds(b * _SL, _SL)
            d = jax.lax.dot_general(ks_parts[a], ks_parts[b],
                                    (((0,), (0,)), ((), ())),
                                    preferred_element_type=jnp.float32)
            m_acc[sla, slb] = m_acc[sla, slb] + d
            if b > a:
                m_acc[slb, sla] = m_acc[slb, sla] + d.T


def kernel(queries, keys, m_persistent):
    t_len, dim = queries.shape
    n_chunks = t_len // _CHUNK
    return pl.pallas_call(
        _qkproj_kernel,
        out_shape=jax.ShapeDtypeStruct((t_len, dim), jnp.float32),
        grid=(n_chunks,),
        in_specs=[
            pl.BlockSpec((_CHUNK, dim), lambda i: (i, 0)),
            pl.BlockSpec((_CHUNK, dim), lambda i: (i, 0)),
            pl.BlockSpec((dim, dim), lambda i: (0, 0)),
        ],
        out_specs=pl.BlockSpec((_CHUNK, dim), lambda i: (i, 0)),
        scratch_shapes=[
            pltpu.SMEM((1, 1), jnp.float32),
        ],
        compiler_params=pltpu.CompilerParams(
            dimension_semantics=("arbitrary",),
        ),
        name="qkprojection",
    )(queries, keys, m_persistent.T)


# speedup vs baseline: 1.0816x; 1.0816x over previous
import jax
import jax.numpy as jnp
from jax.experimental import pallas as pl
from jax.experimental.pallas import tpu as pltpu

_CHUNK = 512
_HALF = 256
_SL = 256
_NORM_PERSISTENT = 1024.0


def _qkproj_kernel(q_ref, k_ref, m_acc, out_ref, n_acc):
    i = pl.program_id(0)

    @pl.when(i == 0)
    def _init():
        n_acc[0, 0] = _NORM_PERSISTENT

    dim = q_ref.shape[1]
    n_sl = dim // _SL
    cdims = (((1,), (1,)), ((), ()))
    rdims = (((1,), (0,)), ((), ()))

    q = q_ref[...]
    k = k_ref[...]
    q0, q1 = q[:_HALF], q[_HALF:]
    k0, k1 = k[:_HALF], k[_HALF:]

    out = jax.lax.dot_general(q, m_acc[...], cdims,
                              preferred_element_type=jnp.float32)

    s00 = jax.lax.dot_general(q0, k0, cdims,
                              preferred_element_type=jnp.float32)
    s10 = jax.lax.dot_general(q1, k0, cdims,
                              preferred_element_type=jnp.float32)
    s11 = jax.lax.dot_general(q1, k1, cdims,
                              preferred_element_type=jnp.float32)
    row = jax.lax.broadcasted_iota(jnp.int32, (_HALF, _HALF), 0)
    col = jax.lax.broadcasted_iota(jnp.int32, (_HALF, _HALF), 1)
    causal = (col <= row)
    s00 = jnp.where(causal, s00, 0.0)
    s11 = jnp.where(causal, s11, 0.0)

    ss = jnp.sum(k * k, axis=1, keepdims=True)
    causal_f = causal.astype(jnp.float32)
    ss0, ss1 = ss[:_HALF], ss[_HALF:]
    csum0 = jnp.dot(causal_f, ss0, preferred_element_type=jnp.float32)
    csum1 = (jnp.sum(ss0) + jnp.dot(causal_f, ss1,
                                    preferred_element_type=jnp.float32))
    base = n_acc[0, 0]
    norms0 = jnp.maximum(base + csum0, 1e-8)
    norms1 = jnp.maximum(base + csum1, 1e-8)
    n_acc[0, 0] = base + jnp.sum(ss)

    d0 = jax.lax.dot_general(s00, k0, rdims,
                             preferred_element_type=jnp.float32)
    d1 = (jax.lax.dot_general(s10, k0, rdims,
                              preferred_element_type=jnp.float32)
          + jax.lax.dot_general(s11, k1, rdims,
                                preferred_element_type=jnp.float32))
    out_ref[:_HALF, :] = (out[:_HALF] + d0) * (1.0 / norms0)
    out_ref[_HALF:, :] = (out[_HALF:] + d1) * (1.0 / norms1)

    ks_parts = [k[:, a * _SL:(a + 1) * _SL] for a in range(n_sl)]
    for a in range(n_sl):
        sla = pl.ds(a * _SL, _SL)
        for b in range(a, n_sl):
            slb = pl.ds(b * _SL, _SL)
            d = jax.lax.dot_general(ks_parts[a], ks_parts[b],
                                    (((0,), (0,)), ((), ())),
                                    preferred_element_type=jnp.float32)
            m_acc[sla, slb] = m_acc[sla, slb] + d
            if b > a:
                m_acc[slb, sla] = m_acc[slb, sla] + d.T


def kernel(queries, keys, m_persistent):
    t_len, dim = queries.shape
    n_chunks = t_len // _CHUNK
    return pl.pallas_call(
        _qkproj_kernel,
        out_shape=jax.ShapeDtypeStruct((t_len, dim), jnp.float32),
        grid=(n_chunks,),
        in_specs=[
            pl.BlockSpec((_CHUNK, dim), lambda i: (i, 0)),
            pl.BlockSpec((_CHUNK, dim), lambda i: (i, 0)),
            pl.BlockSpec((dim, dim), lambda i: (0, 0)),
        ],
        out_specs=pl.BlockSpec((_CHUNK, dim), lambda i: (i, 0)),
        scratch_shapes=[
            pltpu.SMEM((1, 1), jnp.float32),
        ],
        compiler_params=pltpu.CompilerParams(
            dimension_semantics=("arbitrary",),
        ),
        name="qkprojection",
    )(queries, keys, m_persistent)
